# Initial kernel scaffold; baseline (speedup 1.0000x reference)
import jax
import jax.numpy as jnp
from jax.experimental import pallas as pl


def kernel(input_ids, state_table, species_table):
    raise NotImplementedError("write your pallas kernel here")


# speedup vs baseline: 3.7813x; 3.7813x over previous
import jax
import jax.numpy as jnp
from jax.experimental import pallas as pl

B, S, T, H, NUM_STATES = 8, 1000, 24, 256, 3
BS = 40


def _body(ids_ref, state_ref, species_ref, out_ref):
    ids = ids_ref[0][..., None]
    st0 = state_ref[0][None, None, :]
    st1 = state_ref[1][None, None, :]
    st2 = state_ref[2][None, None, :]
    gathered = jnp.where(ids == 0, st0, jnp.where(ids == 1, st1, st2))
    out_ref[0] = gathered + species_ref[...][:, None, :]


def kernel(input_ids, state_table, species_table):
    grid = (B, S // BS)
    return pl.pallas_call(
        _body,
        grid=grid,
        in_specs=[
            pl.BlockSpec((1, BS, T), lambda b, s: (b, s, 0)),
            pl.BlockSpec((NUM_STATES, H), lambda b, s: (0, 0)),
            pl.BlockSpec((BS, H), lambda b, s: (s, 0)),
        ],
        out_specs=pl.BlockSpec((1, BS, T, H), lambda b, s: (b, s, 0, 0)),
        out_shape=jax.ShapeDtypeStruct((B, S, T, H), jnp.float32),
    )(input_ids, state_table, species_table)


# speedup vs baseline: 7.7921x; 2.0607x over previous
import jax
import jax.numpy as jnp
from jax.experimental import pallas as pl

B, S, T, H, NUM_STATES = 8, 1000, 24, 256, 3
BS = 200


def _body(ids_ref, state_ref, species_ref, out_ref):
    ids = ids_ref[0][..., None]
    st0 = state_ref[0][None, None, :]
    st1 = state_ref[1][None, None, :]
    st2 = state_ref[2][None, None, :]
    gathered = jnp.where(ids == 0, st0, jnp.where(ids == 1, st1, st2))
    out_ref[0] = gathered + species_ref[...][:, None, :]


def kernel(input_ids, state_table, species_table):
    grid = (B, S // BS)
    return pl.pallas_call(
        _body,
        grid=grid,
        in_specs=[
            pl.BlockSpec((1, BS, T), lambda b, s: (b, s, 0)),
            pl.BlockSpec((NUM_STATES, H), lambda b, s: (0, 0)),
            pl.BlockSpec((BS, H), lambda b, s: (s, 0)),
        ],
        out_specs=pl.BlockSpec((1, BS, T, H), lambda b, s: (b, s, 0, 0)),
        out_shape=jax.ShapeDtypeStruct((B, S, T, H), jnp.float32),
    )(input_ids, state_table, species_table)


# speedup vs baseline: 8.4743x; 1.0876x over previous
import jax
import jax.numpy as jnp
from jax.experimental import pallas as pl

B, S, T, H, NUM_STATES = 8, 1000, 24, 256, 3
BS = 1000


def _body(ids_ref, state_ref, species_ref, out_ref):
    ids = ids_ref[0][..., None]
    st0 = state_ref[0][None, None, :]
    st1 = state_ref[1][None, None, :]
    st2 = state_ref[2][None, None, :]
    gathered = jnp.where(ids == 0, st0, jnp.where(ids == 1, st1, st2))
    out_ref[0] = gathered + species_ref[...][:, None, :]


def kernel(input_ids, state_table, species_table):
    grid = (B, S // BS)
    return pl.pallas_call(
        _body,
        grid=grid,
        in_specs=[
            pl.BlockSpec((1, BS, T), lambda b, s: (b, s, 0)),
            pl.BlockSpec((NUM_STATES, H), lambda b, s: (0, 0)),
            pl.BlockSpec((BS, H), lambda b, s: (s, 0)),
        ],
        out_specs=pl.BlockSpec((1, BS, T, H), lambda b, s: (b, s, 0, 0)),
        out_shape=jax.ShapeDtypeStruct((B, S, T, H), jnp.float32),
    )(input_ids, state_table, species_table)
